# baseline (speedup 1.0000x reference)
import functools

import jax
import jax.numpy as jnp
from jax import lax
from jax.experimental import pallas as pl
from jax.experimental.pallas import tpu as pltpu
from jax.experimental.pallas import tpu_sc as plsc

F = 100000
L = 16
UNROLL = 8


@functools.lru_cache(maxsize=None)
def _build_sc_kernel():
    info = plsc.get_sparse_core_info()
    ns = info.num_subcores
    nc = 1
    nw = nc * ns
    step = L * UNROLL
    chunk = ((F + nw - 1) // nw + step - 1) // step * step
    n_outer = chunk // step
    mesh = plsc.VectorSubcoreMesh(core_axis_name="c", subcore_axis_name="s",
                                  num_cores=nc, num_subcores=ns)

    @functools.partial(
        pl.kernel,
        out_type=jax.ShapeDtypeStruct((F,), jnp.float32),
        mesh=mesh,
        compiler_params=pltpu.CompilerParams(needs_layout_passes=False),
        scratch_types=[
            pltpu.VMEM((chunk,), jnp.float32),
            pltpu.VMEM((chunk,), jnp.float32),
            pltpu.VMEM((chunk,), jnp.float32),
            pltpu.VMEM((chunk,), jnp.float32),
            pltpu.VMEM((4 * L,), jnp.float32),
            pltpu.SemaphoreType.DMA,
            pltpu.SemaphoreType.DMA,
            pltpu.SemaphoreType.DMA,
        ],
    )
    def sc_kernel(x0_hbm, x1_hbm, x2_hbm, params_hbm, out_hbm,
                  x0_v, x1_v, x2_v, o_v, p_v, s0, s1, s2):
        wid = lax.axis_index("s") * nc + lax.axis_index("c")
        base = jnp.minimum(wid * chunk, F - chunk)
        half = chunk // 2
        cp = pltpu.async_copy(params_hbm, p_v, s0)
        h0 = [pltpu.async_copy(x_hbm.at[pl.ds(base, half)],
                               x_v.at[pl.ds(0, half)], s0)
              for x_hbm, x_v in ((x0_hbm, x0_v), (x1_hbm, x1_v), (x2_hbm, x2_v))]
        h1 = [pltpu.async_copy(x_hbm.at[pl.ds(base + half, half)],
                               x_v.at[pl.ds(half, half)], s1)
              for x_hbm, x_v in ((x0_hbm, x0_v), (x1_hbm, x1_v), (x2_hbm, x2_v))]
        cp.wait()
        for c in h0:
            c.wait()
        w0 = p_v[pl.ds(0 * L, L)]
        w1 = p_v[pl.ds(1 * L, L)]
        w2 = p_v[pl.ds(2 * L, L)]
        cc = p_v[pl.ds(3 * L, L)]

        def bf16_round(v):
            u = plsc.bitcast(v, jnp.int32)
            u = (u + 0x7FFF + ((u >> 16) & 1)) & jnp.int32(-65536)
            return plsc.bitcast(u, jnp.float32)

        LOG2E = 1.4426950408889634
        LN2_HI = 0.693359375
        LN2_LO = -2.12194440e-4

        def compute(lo, hi):
            @plsc.parallel_loop(lo, hi, step=L, unroll=UNROLL)
            def _loop(off):
                a0 = x0_v[pl.ds(off, L)]
                a1 = x1_v[pl.ds(off, L)]
                a2 = x2_v[pl.ds(off, L)]
                x = (w0 * bf16_round(a0) + w1 * bf16_round(a1)
                     + w2 * bf16_round(a2) + cc)
                bias = jnp.where(x >= 0.0, 0.5, -0.5)
                n = jnp.clip((x * LOG2E + bias).astype(jnp.int32), -126, 126)
                nf = n.astype(jnp.float32)
                r = (x - nf * LN2_HI) - nf * LN2_LO
                scale = plsc.bitcast((n + 127) << 23, jnp.float32)
                e = jnp.exp(r) * scale
                o_v[pl.ds(off, L)] = (a0 + a1 + a2) / (1.0 + e)

        compute(0, half)
        o0 = pltpu.async_copy(o_v.at[pl.ds(0, half)],
                              out_hbm.at[pl.ds(base, half)], s2)
        for c in h1:
            c.wait()
        compute(half, chunk)
        o1 = pltpu.async_copy(o_v.at[pl.ds(half, half)],
                              out_hbm.at[pl.ds(base + half, half)], s2)
        o0.wait()
        o1.wait()

    return sc_kernel


def kernel(intensities, avg_dist, W_gcn, b_gcn, W_attn, b_attn):
    del W_gcn, b_gcn
    def _bf16_round(v):
        u = lax.bitcast_convert_type(v, jnp.int32)
        u = (u + 0x7FFF + ((u >> 16) & 1)) & jnp.int32(-65536)
        return lax.bitcast_convert_type(u, jnp.float32)

    wb = _bf16_round(W_attn[0])
    avb = _bf16_round(avg_dist)
    w = -wb
    c = w[3] * avb - b_attn[0]
    params = jnp.concatenate([
        jnp.full((L,), w[0], jnp.float32),
        jnp.full((L,), w[1], jnp.float32),
        jnp.full((L,), w[2], jnp.float32),
        jnp.full((L,), c, jnp.float32),
    ])
    return _build_sc_kernel()(intensities[0], intensities[1], intensities[2],
                              params)
